# baseline (speedup 1.0000x reference)
import jax
import jax.numpy as jnp
from jax import lax
from jax.experimental import pallas as pl
from jax.experimental.pallas import tpu as pltpu
from jax.experimental.pallas import tpu_sc as plsc

_B = 16384
_D = 64
_V = 1000000
_NC = 2
_NS = 16
_NW = _NC * _NS
_BPW = _B // _NW
_L = 16
_NIDX = _BPW * _D
_NBANK = _BPW // _L
_BANKIDX = _L * _D
_SCALE = 0.5 * 0.5 / _B


def _sc_body(x_hbm, lab_hbm, cflat_hbm, out_hbm,
             lab_v, idx_v, x_v, bufa, bufb, acc_v, sem_a, sem_b):
    wid = lax.axis_index("s") * _NC + lax.axis_index("c")
    iota = lax.iota(jnp.int32, _L)
    rvec = [_V * (g * _L + iota) for g in range(_D // _L)]

    pltpu.sync_copy(lab_hbm.at[pl.ds(wid * _BPW, _BPW)], lab_v)
    pltpu.sync_copy(x_hbm.at[pl.ds(wid * _BPW * _D, _BPW * _D)], x_v)

    def build(m, carry):
        lab16 = lab_v[pl.ds(m * _L, _L)]
        for k in range(_L):
            base = lab16[k]
            for g in range(_D // _L):
                idx_v[pl.ds(m * _BANKIDX + k * _D + g * _L, _L)] = rvec[g] + base
        return carry

    lax.fori_loop(0, _NBANK, build, jnp.int32(0))

    def issue(m, buf, sem):
        mm = jnp.minimum(m, _NBANK - 1)
        for j in range(8):
            pltpu.async_copy(
                cflat_hbm.at[idx_v.at[pl.ds(mm * _BANKIDX + j * 128, 128)]],
                buf.at[pl.ds(j * 128, 128)], sem)

    def drain(buf, sem):
        for j in range(8):
            pltpu.make_async_copy(cflat_hbm.at[pl.ds(0, 128)],
                                  buf.at[pl.ds(j * 128, 128)], sem).wait()

    def process(m, buf, acc):
        for k in range(_L):
            for g in range(_D // _L):
                xv = x_v[pl.ds(m * _BANKIDX + k * _D + g * _L, _L)]
                cv = buf[pl.ds(k * _D + g * _L, _L)]
                d = xv - cv
                acc = acc + d * d
        return acc

    issue(jnp.int32(0), bufa, sem_a)
    issue(jnp.int32(1), bufb, sem_b)

    def pair(j, acc):
        drain(bufa, sem_a)
        acc = process(2 * j, bufa, acc)
        issue(2 * j + 2, bufa, sem_a)
        drain(bufb, sem_b)
        acc = process(2 * j + 1, bufb, acc)
        issue(2 * j + 3, bufb, sem_b)
        return acc

    acc = lax.fori_loop(0, _NBANK // 2, pair, jnp.zeros((_L,), jnp.float32))
    drain(bufa, sem_a)
    drain(bufb, sem_b)

    acc_v[...] = acc
    pltpu.sync_copy(acc_v, out_hbm.at[wid])


@jax.jit
def _center_loss(x, labels_i32, centers):
    mesh = plsc.VectorSubcoreMesh(core_axis_name="c", subcore_axis_name="s")
    cflat = centers.T.reshape(-1)
    partials = pl.kernel(
        _sc_body,
        out_type=jax.ShapeDtypeStruct((_NW, _L), jnp.float32),
        mesh=mesh,
        scratch_types=[
            pltpu.VMEM((_BPW,), jnp.int32),
            pltpu.VMEM((_NIDX,), jnp.int32),
            pltpu.VMEM((_BPW * _D,), jnp.float32),
            pltpu.VMEM((_BANKIDX,), jnp.float32),
            pltpu.VMEM((_BANKIDX,), jnp.float32),
            pltpu.VMEM((_L,), jnp.float32),
            pltpu.SemaphoreType.DMA,
            pltpu.SemaphoreType.DMA,
        ],
    )(x.reshape(-1), labels_i32, cflat)
    return _SCALE * jnp.sum(partials)


def kernel(x, labels, centers):
    return _center_loss(x, labels.astype(jnp.int32), centers)


# speedup vs baseline: 1.0046x; 1.0046x over previous
import jax
import jax.numpy as jnp
from jax import lax
from jax.experimental import pallas as pl
from jax.experimental.pallas import tpu as pltpu
from jax.experimental.pallas import tpu_sc as plsc

_B = 16384
_D = 64
_V = 1000000
_NC = 2
_NS = 16
_NW = _NC * _NS
_BPW = _B // _NW
_L = 16
_NIDX = _BPW * _D
_NBANK = _BPW // _L
_BANKIDX = _L * _D
_SCALE = 0.5 * 0.5 / _B


def _sc_body(x_hbm, lab_hbm, cflat_hbm, out_hbm,
             lab_v, idx_v, x_v, bufa, bufb, acc_v, sem_a, sem_b):
    wid = lax.axis_index("s") * _NC + lax.axis_index("c")
    iota = lax.iota(jnp.int32, _L)
    rvec = [_V * (g * _L + iota) for g in range(_D // _L)]

    pltpu.sync_copy(lab_hbm.at[pl.ds(wid * _BPW, _BPW)], lab_v)
    pltpu.sync_copy(x_hbm.at[pl.ds(wid * _BPW * _D, _BPW * _D)], x_v)

    def build(m, carry):
        lab16 = lab_v[pl.ds(m * _L, _L)]
        for k in range(_L):
            base = lab16[k]
            for g in range(_D // _L):
                idx_v[pl.ds(m * _BANKIDX + k * _D + g * _L, _L)] = rvec[g] + base
        return carry

    lax.fori_loop(0, _NBANK, build, jnp.int32(0))

    def issue(m, buf, sem):
        mm = jnp.minimum(m, _NBANK - 1)
        pltpu.async_copy(
            cflat_hbm.at[idx_v.at[pl.ds(mm * _BANKIDX, _BANKIDX)]], buf, sem)

    def drain(buf, sem):
        pltpu.make_async_copy(cflat_hbm.at[pl.ds(0, _BANKIDX)], buf,
                              sem).wait()

    def process(m, buf, acc):
        for k in range(_L):
            for g in range(_D // _L):
                xv = x_v[pl.ds(m * _BANKIDX + k * _D + g * _L, _L)]
                cv = buf[pl.ds(k * _D + g * _L, _L)]
                d = xv - cv
                acc = acc + d * d
        return acc

    issue(jnp.int32(0), bufa, sem_a)
    issue(jnp.int32(1), bufb, sem_b)

    def pair(j, acc):
        drain(bufa, sem_a)
        acc = process(2 * j, bufa, acc)
        issue(2 * j + 2, bufa, sem_a)
        drain(bufb, sem_b)
        acc = process(2 * j + 1, bufb, acc)
        issue(2 * j + 3, bufb, sem_b)
        return acc

    acc = lax.fori_loop(0, _NBANK // 2, pair, jnp.zeros((_L,), jnp.float32))
    drain(bufa, sem_a)
    drain(bufb, sem_b)

    acc_v[...] = acc
    pltpu.sync_copy(acc_v, out_hbm.at[wid])


@jax.jit
def _center_loss(x, labels_i32, centers):
    mesh = plsc.VectorSubcoreMesh(core_axis_name="c", subcore_axis_name="s")
    cflat = centers.T.reshape(-1)
    partials = pl.kernel(
        _sc_body,
        out_type=jax.ShapeDtypeStruct((_NW, _L), jnp.float32),
        mesh=mesh,
        scratch_types=[
            pltpu.VMEM((_BPW,), jnp.int32),
            pltpu.VMEM((_NIDX,), jnp.int32),
            pltpu.VMEM((_BPW * _D,), jnp.float32),
            pltpu.VMEM((_BANKIDX,), jnp.float32),
            pltpu.VMEM((_BANKIDX,), jnp.float32),
            pltpu.VMEM((_L,), jnp.float32),
            pltpu.SemaphoreType.DMA,
            pltpu.SemaphoreType.DMA,
        ],
    )(x.reshape(-1), labels_i32, cflat)
    return _SCALE * jnp.sum(partials)


def kernel(x, labels, centers):
    return _center_loss(x, labels.astype(jnp.int32), centers)


# speedup vs baseline: 3.1922x; 3.1775x over previous
import functools

import jax
import jax.numpy as jnp
from jax import lax
from jax.experimental import pallas as pl
from jax.experimental.pallas import tpu as pltpu

_B = 16384
_D = 64
_V = 1000000
_W = 512
_NWIN = -(-_V // _W)
_K = 8
_SCALE = 0.5 * 0.5 / _B


def _win_body(starts_ref, xl_ref, c_ref, out_ref):
    t = pl.program_id(0)
    c = c_ref[...]
    nwin = jnp.sum(c * c, axis=0, keepdims=True)
    lanef = (t * _W
             + lax.broadcasted_iota(jnp.int32, (1, _W), 1)).astype(jnp.float32)

    gs = starts_ref[t]
    ge = starts_ref[t + 1]
    nch = (ge - gs + _K - 1) // _K

    def chunk(i, acc):
        rs = gs + i * _K
        rs_c = jnp.minimum(rs, _B - _K)
        xs = xl_ref[pl.ds(rs_c, _K), 0:_D]
        labf = xl_ref[pl.ds(rs_c, _K), _D:_D + 1]
        pos = rs_c + lax.broadcasted_iota(jnp.int32, (_K, 1), 0)
        valid = (pos >= rs) & (pos < ge)
        oh = (labf == lanef) & valid
        g = jnp.dot(xs, c, preferred_element_type=jnp.float32)
        return acc + jnp.sum(jnp.where(oh, nwin - 2.0 * g, 0.0))

    acc = lax.fori_loop(0, nch, chunk, jnp.float32(0.0))

    @pl.when(t == 0)
    def _():
        xall = xl_ref[:, 0:_D]
        out_ref[...] = jnp.full((1, 8, 128), acc + jnp.sum(xall * xall),
                                jnp.float32)

    @pl.when(t != 0)
    def _():
        out_ref[...] = jnp.full((1, 8, 128), acc, jnp.float32)


@jax.jit
def _center_loss(x, labels_i32, centers):
    order = jnp.argsort(labels_i32)
    labs = labels_i32[order]
    xs = x[order]
    xl = jnp.concatenate(
        [xs, labs.astype(jnp.float32)[:, None],
         jnp.zeros((_B, 128 - _D - 1), jnp.float32)], axis=1)
    bounds = jnp.arange(_NWIN + 1, dtype=jnp.int32) * _W
    starts = jnp.searchsorted(labs, bounds).astype(jnp.int32)

    partials = pl.pallas_call(
        _win_body,
        grid_spec=pltpu.PrefetchScalarGridSpec(
            num_scalar_prefetch=1,
            grid=(_NWIN,),
            in_specs=[
                pl.BlockSpec((_B, 128), lambda t, s: (0, 0)),
                pl.BlockSpec((_D, _W), lambda t, s: (0, t)),
            ],
            out_specs=pl.BlockSpec((1, 8, 128), lambda t, s: (t, 0, 0)),
        ),
        out_shape=jax.ShapeDtypeStruct((_NWIN, 8, 128), jnp.float32),
    )(starts, xl, centers.T)
    return _SCALE * jnp.sum(partials[:, 0, 0])


def kernel(x, labels, centers):
    return _center_loss(x, labels.astype(jnp.int32), centers)
